# baseline (speedup 1.0000x reference)
import functools

import jax
import jax.numpy as jnp
from jax import lax
from jax.experimental import pallas as pl
from jax.experimental.pallas import tpu as pltpu
from jax.experimental.pallas import tpu_sc as plsc

_NUM_FIELDS = 26
_FIELD_DIM = 100000
_BATCH = 4096
_EMB = 16
_IN_DIM = _NUM_FIELDS * _EMB
_EPS_BN = 1e-5


def _sc_gather(X, emb_table, lin_flat):
    info = plsc.get_sparse_core_info()
    nc, ns = info.num_cores, info.num_subcores
    nw = nc * ns
    bpw = _BATCH // nw

    mesh = plsc.VectorSubcoreMesh(core_axis_name="c", subcore_axis_name="s")

    @functools.partial(
        pl.kernel,
        out_type=(
            jax.ShapeDtypeStruct((_BATCH, _IN_DIM), jnp.float32),
            jax.ShapeDtypeStruct((_BATCH,), jnp.float32),
        ),
        mesh=mesh,
        compiler_params=pltpu.CompilerParams(use_tc_tiling_on_sc=False),
        scratch_types=[
            pltpu.VMEM((_NUM_FIELDS, bpw), jnp.int32),
            pltpu.VMEM((_NUM_FIELDS, bpw), jnp.int32),
            pltpu.VMEM((_NUM_FIELDS, bpw, _EMB), jnp.float32),
            pltpu.VMEM((_NUM_FIELDS, bpw), jnp.float32),
            pltpu.VMEM((bpw,), jnp.float32),
            pltpu.SemaphoreType.DMA,
            pltpu.SemaphoreType.DMA,
            pltpu.SemaphoreType.DMA,
        ],
    )
    def k(x_hbm, emb_hbm, lin_hbm, h_out, lin_out,
          x_v, idx_v, rows_v, lv_v, acc_v, gsem, lsem, osem):
        wid = lax.axis_index("s") * nc + lax.axis_index("c")
        b0 = wid * bpw
        pltpu.sync_copy(x_hbm.at[:, pl.ds(b0, bpw)], x_v)

        for f in range(_NUM_FIELDS):
            @pl.loop(0, bpw, step=16)
            def _(g, f=f):
                idx_v[f, pl.ds(g, 16)] = x_v[f, pl.ds(g, 16)] + f * _FIELD_DIM

        gathers = []
        for f in range(_NUM_FIELDS):
            gathers.append(
                pltpu.async_copy(emb_hbm.at[idx_v.at[f]], rows_v.at[f], gsem))
            gathers.append(
                pltpu.async_copy(lin_hbm.at[idx_v.at[f]], lv_v.at[f], lsem))
        for cp in gathers:
            cp.wait()

        writes = []
        for f in range(_NUM_FIELDS):
            writes.append(pltpu.async_copy(
                rows_v.at[f],
                h_out.at[pl.ds(b0, bpw), pl.ds(f * _EMB, _EMB)],
                osem))

        @pl.loop(0, bpw, step=16)
        def _(g):
            acc = lv_v[0, pl.ds(g, 16)]
            for f in range(1, _NUM_FIELDS):
                acc = acc + lv_v[f, pl.ds(g, 16)]
            acc_v[pl.ds(g, 16)] = acc

        pltpu.sync_copy(acc_v, lin_out.at[pl.ds(b0, bpw)])
        for cp in writes:
            cp.wait()

    return k(X, emb_table, lin_flat)


def _tc_mlp(h, lin, y, W1, b1, g1, bt1, W2, b2, g2, bt2, w3row, b3, lin_bias, S):
    def body(h_ref, lin_ref, y_ref, w1_ref, b1_ref, g1_ref, bt1_ref,
             w2_ref, b2_ref, g2_ref, bt2_ref, w3_ref, b3_ref, lb_ref,
             s_ref, out_ref):
        hv = h_ref[...]
        sm = s_ref[...]
        s = jnp.dot(hv, sm, preferred_element_type=jnp.float32)
        ss = jnp.dot(hv * hv, sm, preferred_element_type=jnp.float32)
        fm = 0.5 * jnp.sum(s * s - ss, axis=1, keepdims=True)

        z1 = jnp.dot(hv, w1_ref[...], preferred_element_type=jnp.float32)
        z1 = z1 + b1_ref[...]
        m1 = jnp.mean(z1, axis=0, keepdims=True)
        v1 = jnp.mean(z1 * z1, axis=0, keepdims=True) - m1 * m1
        a1 = jnp.maximum(
            g1_ref[...] * (z1 - m1) * lax.rsqrt(v1 + _EPS_BN) + bt1_ref[...], 0.0)

        z2 = jnp.dot(a1, w2_ref[...], preferred_element_type=jnp.float32)
        z2 = z2 + b2_ref[...]
        m2 = jnp.mean(z2, axis=0, keepdims=True)
        v2 = jnp.mean(z2 * z2, axis=0, keepdims=True) - m2 * m2
        a2 = jnp.maximum(
            g2_ref[...] * (z2 - m2) * lax.rsqrt(v2 + _EPS_BN) + bt2_ref[...], 0.0)

        z3 = jnp.sum(a2 * w3_ref[...], axis=1, keepdims=True) + b3_ref[...]
        logits = z3 + lin_ref[...] + lb_ref[...] + fm
        p = 1.0 / (1.0 + jnp.exp(-logits))
        p = jnp.clip(p, 1e-7, 1.0 - 1e-7)
        yv = y_ref[...]
        ll = yv * jnp.log(p) + (1.0 - yv) * jnp.log(1.0 - p)
        out_ref[...] = jnp.reshape(-jnp.sum(ll) * (1.0 / _BATCH), (1, 1))

    return pl.pallas_call(
        body,
        out_shape=jax.ShapeDtypeStruct((1, 1), jnp.float32),
    )(h, lin, y, W1, b1, g1, bt1, W2, b2, g2, bt2, w3row, b3, lin_bias, S)


def kernel(X, y, emb_table, lin_table, lin_bias, W1, b1, g1, bt1,
           W2, b2, g2, bt2, W3, b3):
    h, lin = _sc_gather(X, emb_table, lin_table.reshape(-1))
    S = jnp.tile(jnp.eye(_EMB, dtype=jnp.float32), (_NUM_FIELDS, 1))
    loss = _tc_mlp(
        h, lin.reshape(_BATCH, 1), y,
        W1, b1.reshape(1, -1), g1.reshape(1, -1), bt1.reshape(1, -1),
        W2, b2.reshape(1, -1), g2.reshape(1, -1), bt2.reshape(1, -1),
        W3.reshape(1, -1), b3.reshape(1, 1), lin_bias.reshape(1, 1), S)
    return loss[0, 0]
